# baseline (speedup 1.0000x reference)
import jax
import jax.numpy as jnp
from jax.experimental import pallas as pl
from jax.experimental.pallas import tpu as pltpu

B_ = 1024
S_ = 50
NUM_CLASSES_ = 1000
NBLK_ = 64
R_ = B_ // NBLK_
K_ = 6


def _onehot_body(x_ref, o_hbm, scratch, sems):
    ids = jax.lax.broadcasted_iota(jnp.int32, (R_, S_, NUM_CLASSES_), 2)

    def step(i, _):
        slot = jax.lax.rem(i, K_)

        @pl.when(i >= K_)
        def _wait_prev():
            pltpu.make_async_copy(
                scratch.at[slot],
                o_hbm.at[pl.ds((i - K_) * R_, R_)],
                sems.at[slot],
            ).wait()

        xv = x_ref[pl.ds(i * R_, R_), :]
        scratch[slot] = (ids == xv[:, :, None]).astype(scratch.dtype)
        pltpu.make_async_copy(
            scratch.at[slot],
            o_hbm.at[pl.ds(i * R_, R_)],
            sems.at[slot],
        ).start()
        return 0

    jax.lax.fori_loop(0, NBLK_, step, 0)

    def drain(j, _):
        i = NBLK_ - K_ + j
        slot = jax.lax.rem(i, K_)
        pltpu.make_async_copy(
            scratch.at[slot],
            o_hbm.at[pl.ds(i * R_, R_)],
            sems.at[slot],
        ).wait()
        return 0

    jax.lax.fori_loop(0, K_, drain, 0)


def kernel(x):
    out_dtype = jnp.zeros((), jnp.int64).dtype
    x = x.astype(jnp.int32)
    return pl.pallas_call(
        _onehot_body,
        in_specs=[pl.BlockSpec(memory_space=pltpu.MemorySpace.VMEM)],
        out_specs=pl.BlockSpec(memory_space=pltpu.MemorySpace.HBM),
        out_shape=jax.ShapeDtypeStruct((B_, S_, NUM_CLASSES_), out_dtype),
        scratch_shapes=[
            pltpu.MemorySpace.VMEM((K_, R_, S_, NUM_CLASSES_), jnp.int32),
            pltpu.SemaphoreType.DMA((K_,)),
        ],
        compiler_params=pltpu.CompilerParams(
            vmem_limit_bytes=100 * 1024 * 1024,
        ),
    )(x)


# speedup vs baseline: 1.0247x; 1.0247x over previous
import jax
import jax.numpy as jnp
from jax.experimental import pallas as pl
from jax.experimental.pallas import tpu as pltpu

NUM_CLASSES_ = 1000
ROWS_PER_BLOCK = 64


def _onehot_block(x_ref, o_ref):
    ids = jax.lax.broadcasted_iota(jnp.int32, o_ref.shape, 2)
    xv = x_ref[...]
    o_ref[...] = (ids == xv[:, :, None]).astype(o_ref.dtype)


def kernel(x):
    out_dtype = jnp.zeros((), jnp.int64).dtype
    b, s = x.shape
    x = x.astype(jnp.int32)
    grid = b // ROWS_PER_BLOCK
    return pl.pallas_call(
        _onehot_block,
        grid=(grid,),
        in_specs=[pl.BlockSpec((ROWS_PER_BLOCK, s), lambda i: (i, 0))],
        out_specs=pl.BlockSpec(
            (ROWS_PER_BLOCK, s, NUM_CLASSES_), lambda i: (i, 0, 0)
        ),
        out_shape=jax.ShapeDtypeStruct((b, s, NUM_CLASSES_), out_dtype),
        compiler_params=pltpu.CompilerParams(
            dimension_semantics=("parallel",),
        ),
    )(x)
